# baseline (speedup 1.0000x reference)
import functools

import jax
import jax.numpy as jnp
from jax import lax
from jax.experimental import pallas as pl
from jax.experimental.pallas import tpu as pltpu
from jax.experimental.pallas import tpu_sc as plsc

MODEL_DIM = 64
NUM_CORES = 2
NUM_SUBCORES = 16
NUM_WORKERS = NUM_CORES * NUM_SUBCORES
BLK = 128
SCALE = 8.0


def _sc_embedding_lookup(table, idx3):
    chunks_per_w = idx3.shape[1]
    n_chunks_total = NUM_WORKERS * chunks_per_w
    seq = 50
    n_blocks = n_chunks_total // seq
    batch = n_blocks * BLK

    mesh = plsc.VectorSubcoreMesh(core_axis_name="c", subcore_axis_name="s")

    @functools.partial(
        pl.kernel,
        mesh=mesh,
        out_type=jax.ShapeDtypeStruct((seq, MODEL_DIM, batch), jnp.float32),
        scratch_types=[
            pltpu.VMEM((chunks_per_w, BLK), jnp.int32),
            pltpu.VMEM((BLK, MODEL_DIM), jnp.float32),
            pltpu.VMEM((BLK, MODEL_DIM), jnp.float32),
            pltpu.SemaphoreType.DMA,
            pltpu.SemaphoreType.DMA,
            pltpu.SemaphoreType.DMA,
            pltpu.SemaphoreType.DMA,
            pltpu.SemaphoreType.DMA,
        ],
        compiler_params=pltpu.CompilerParams(needs_layout_passes=False),
    )
    def k(table_hbm, idx_hbm, out_hbm, idx_v, rows_a, rows_b,
          g_a, g_b, o_a, o_b, s_s):
        cid = lax.axis_index("c")
        sid = lax.axis_index("s")
        wid = sid * NUM_CORES + cid
        gbase = wid * chunks_per_w
        pltpu.async_copy(idx_hbm.at[wid], idx_v, s_s).wait()

        def start_rows(c, rows_ref, sem):
            for t0 in range(0, BLK, 16):
                v = idx_v[c, pl.ds(t0, 16)]
                for j in range(16):
                    rid = v[j]
                    pltpu.async_copy(table_hbm.at[rid], rows_ref.at[t0 + j], sem)

        def wait_rows(rows_ref, sem):
            for t in range(BLK):
                pltpu.make_async_copy(
                    table_hbm.at[0], rows_ref.at[t], sem
                ).wait()

        def scale_rows(rows_ref):
            for t in range(BLK):
                for d0 in range(MODEL_DIM // 16):
                    sl = pl.ds(d0 * 16, 16)
                    rows_ref[t, sl] = rows_ref[t, sl] * SCALE

        def start_out(c, rows_ref, sem):
            g = gbase + c
            s = g // n_blocks
            b0 = (g % n_blocks) * BLK
            for d in range(MODEL_DIM):
                pltpu.async_copy(
                    rows_ref.at[:, d],
                    out_hbm.at[s, d, pl.ds(b0, BLK)],
                    sem,
                )

        def wait_out(rows_ref, sem):
            for d in range(MODEL_DIM):
                pltpu.make_async_copy(
                    rows_ref.at[:, d],
                    out_hbm.at[0, d, pl.ds(0, BLK)],
                    sem,
                ).wait()

        start_rows(0, rows_a, g_a)
        start_rows(1, rows_b, g_b)

        def body(i, carry):
            c0 = 2 * i
            wait_rows(rows_a, g_a)
            scale_rows(rows_a)
            start_out(c0, rows_a, o_a)

            @pl.when(i < chunks_per_w // 2 - 1)
            def _():
                wait_out(rows_a, o_a)
                start_rows(c0 + 2, rows_a, g_a)

            wait_rows(rows_b, g_b)
            scale_rows(rows_b)
            start_out(c0 + 1, rows_b, o_b)

            @pl.when(i < chunks_per_w // 2 - 1)
            def _():
                wait_out(rows_b, o_b)
                start_rows(c0 + 3, rows_b, g_b)

            return carry

        lax.fori_loop(0, chunks_per_w // 2, body, 0)
        wait_out(rows_a, o_a)
        wait_out(rows_b, o_b)

    return k(table, idx3)


def kernel(token_indices, embeddings):
    b, s = token_indices.shape
    total = b * s
    chunks_per_w = total // BLK // NUM_WORKERS
    idx3 = token_indices.T.reshape(NUM_WORKERS, chunks_per_w, BLK).astype(jnp.int32)
    outT = _sc_embedding_lookup(embeddings, idx3)
    return outT.transpose(2, 0, 1)


# speedup vs baseline: 118.5117x; 118.5117x over previous
import functools

import jax
import jax.numpy as jnp
from jax import lax
from jax.experimental import pallas as pl
from jax.experimental.pallas import tpu as pltpu
from jax.experimental.pallas import tpu_sc as plsc

MODEL_DIM = 64
NUM_CORES = 2
NUM_SUBCORES = 16
NUM_WORKERS = NUM_CORES * NUM_SUBCORES
BLK = 128
PITCH = 131
SCALE = 8.0


def _sc_embedding_lookup(table, idx3):
    chunks_per_w = idx3.shape[1]
    n_chunks_total = NUM_WORKERS * chunks_per_w
    seq = 50
    n_blocks = n_chunks_total // seq
    batch = n_blocks * BLK

    mesh = plsc.VectorSubcoreMesh(core_axis_name="c", subcore_axis_name="s")

    @functools.partial(
        pl.kernel,
        mesh=mesh,
        out_type=jax.ShapeDtypeStruct((seq, MODEL_DIM, batch), jnp.float32),
        scratch_types=[
            pltpu.VMEM((chunks_per_w, BLK), jnp.int32),
            pltpu.VMEM((BLK, MODEL_DIM), jnp.float32),
            pltpu.VMEM((BLK, MODEL_DIM), jnp.float32),
            pltpu.VMEM((MODEL_DIM, PITCH), jnp.float32),
            pltpu.VMEM((MODEL_DIM, PITCH), jnp.float32),
            pltpu.SemaphoreType.DMA,
            pltpu.SemaphoreType.DMA,
            pltpu.SemaphoreType.DMA,
            pltpu.SemaphoreType.DMA,
        ],
        compiler_params=pltpu.CompilerParams(
            use_tc_tiling_on_sc=False, needs_layout_passes=False
        ),
    )
    def k(table_hbm, idx_hbm, out_hbm, idx_v, rows_a, rows_b, outb_a, outb_b,
          g_a, g_b, o_a, o_b):
        cid = lax.axis_index("c")
        sid = lax.axis_index("s")
        wid = sid * NUM_CORES + cid
        gbase = wid * chunks_per_w
        pltpu.sync_copy(idx_hbm.at[wid], idx_v)

        def start_gather(c, rows_ref, sem):
            pltpu.async_copy(table_hbm.at[idx_v.at[c]], rows_ref, sem)

        def wait_gather(rows_ref, sem):
            pltpu.make_async_copy(
                table_hbm.at[pl.ds(0, BLK)], rows_ref, sem
            ).wait()

        def transpose_scale(rows_ref, outb_ref):
            lane = lax.iota(jnp.int32, 16)

            @plsc.parallel_loop(0, BLK, 1, unroll=8)
            def _(t):
                tvec = jnp.full((16,), t, jnp.int32)
                for d0 in range(MODEL_DIM // 16):
                    dvec = lane + (d0 * 16)
                    vals = rows_ref[t, pl.ds(d0 * 16, 16)]
                    plsc.store_scatter(outb_ref, [dvec, tvec], vals * SCALE)

        def start_out(c, outb_ref, sem):
            g = gbase + c
            s = g // n_blocks
            b0 = (g % n_blocks) * BLK
            pltpu.async_copy(
                outb_ref.at[:, pl.ds(0, BLK)],
                out_hbm.at[s, :, pl.ds(b0, BLK)],
                sem,
            )

        def wait_out(outb_ref, sem):
            pltpu.make_async_copy(
                outb_ref.at[:, pl.ds(0, BLK)],
                out_hbm.at[0, :, pl.ds(0, BLK)],
                sem,
            ).wait()

        start_gather(0, rows_a, g_a)
        start_gather(1, rows_b, g_b)

        def body(i, carry):
            c0 = 2 * i
            wait_gather(rows_a, g_a)

            @pl.when(i > 0)
            def _():
                wait_out(outb_a, o_a)

            transpose_scale(rows_a, outb_a)
            start_out(c0, outb_a, o_a)

            @pl.when(i < chunks_per_w // 2 - 1)
            def _():
                start_gather(c0 + 2, rows_a, g_a)

            wait_gather(rows_b, g_b)

            @pl.when(i > 0)
            def _():
                wait_out(outb_b, o_b)

            transpose_scale(rows_b, outb_b)
            start_out(c0 + 1, outb_b, o_b)

            @pl.when(i < chunks_per_w // 2 - 1)
            def _():
                start_gather(c0 + 3, rows_b, g_b)

            return carry

        lax.fori_loop(0, chunks_per_w // 2, body, 0)
        wait_out(outb_a, o_a)
        wait_out(outb_b, o_b)

    return k(table, idx3)


def kernel(token_indices, embeddings):
    b, s = token_indices.shape
    total = b * s
    chunks_per_w = total // BLK // NUM_WORKERS
    idx3 = token_indices.T.reshape(NUM_WORKERS, chunks_per_w, BLK).astype(jnp.int32)
    outT = _sc_embedding_lookup(embeddings, idx3)
    return outT.transpose(2, 0, 1)
